# Initial kernel scaffold; baseline (speedup 1.0000x reference)
import jax
import jax.numpy as jnp
from jax.experimental import pallas as pl


def kernel(x, gate_w, gate_b, w1, b1, w2, b2):
    raise NotImplementedError("write your pallas kernel here")


# speedup vs baseline: 3.2712x; 3.2712x over previous
import functools

import jax
import jax.numpy as jnp
from jax import lax
from jax.experimental import pallas as pl
from jax.experimental.pallas import tpu as pltpu
from jax.experimental.pallas import tpu_sc as plsc

DIM = 768
DFF = 3072
E = 8
NT = 2048
CAP = 307
CPAD = 320
LPAD = 128
FT = 512
NROWS = E * CPAD

_SC_NC = 2
_SC_NS = 16
_SC_NW = _SC_NC * _SC_NS


def _lane_cumsum(a):
    n = a.shape[-1]
    sh = 1
    while sh < n:
        z = jnp.zeros(a.shape[:-1] + (sh,), a.dtype)
        a = a + jnp.concatenate([z, a[..., : n - sh]], axis=-1)
        sh *= 2
    return a


def _route_kernel(x_ref, gwt_ref, gb_ref, dtok_ref, cidx_ref, cw_ref):
    x = x_ref[...]
    logits = lax.dot_general(
        x, gwt_ref[...], (((1,), (0,)), ((), ())),
        preferred_element_type=jnp.float32,
    ) + gb_ref[...]
    lane = lax.broadcasted_iota(jnp.int32, (NT, LPAD), 1)
    logits = jnp.where(lane < E, logits, -jnp.inf)
    m = jnp.max(logits, axis=1, keepdims=True)
    el = jnp.exp(logits - m)
    p = el / jnp.sum(el, axis=1, keepdims=True)

    m1 = jnp.max(p, axis=1, keepdims=True)
    a1 = jnp.min(jnp.where(p == m1, lane, LPAD), axis=1, keepdims=True)
    p2 = jnp.where(lane == a1, -1.0, p)
    m2 = jnp.max(p2, axis=1, keepdims=True)
    a2 = jnp.min(jnp.where(p2 == m2, lane, LPAD), axis=1, keepdims=True)

    pT = jnp.transpose(p[:, :E])
    a1T = jnp.transpose(a1)
    a2T = jnp.transpose(a2)
    eio = lax.broadcasted_iota(jnp.int32, (E, NT), 0)
    sel = (eio == a1T) | (eio == a2T)

    keys = jnp.where(sel, lax.bitcast_convert_type(pT, jnp.int32), -1)

    lo = jnp.zeros((E, 1), jnp.int32)
    hi = jnp.full((E, 1), 1 << 30, jnp.int32)
    for _ in range(31):
        mid = lo + ((hi - lo + 1) >> 1)
        cnt = jnp.sum((keys >= mid).astype(jnp.int32), axis=1, keepdims=True)
        pred = cnt >= CAP
        lo = jnp.where(pred, mid, lo)
        hi = jnp.where(pred, hi, mid - 1)
    kept = keys >= lo

    cums = _lane_cumsum(kept.astype(jnp.int32))
    slot = cums - 1

    cio = lax.broadcasted_iota(jnp.int32, (CPAD, NT), 0)
    for e in range(E):
        m_le = (cums[e : e + 1, :] <= cio).astype(jnp.int32)
        t_star = jnp.sum(m_le, axis=1)
        dtok_ref[e, :] = jnp.minimum(t_star, NT - 1)

    def pick(aT):
        onehot = eio == aT
        sl = jnp.sum(jnp.where(onehot, slot, 0), axis=0, keepdims=True)
        kp = jnp.sum(jnp.where(onehot & kept, 1, 0), axis=0, keepdims=True) > 0
        wv = jnp.sum(jnp.where(onehot, pT, 0.0), axis=0, keepdims=True)
        gidx = aT * CPAD + sl
        return jnp.where(kp, gidx, 0), jnp.where(kp, wv, 0.0)

    ci0, cw0 = pick(a1T)
    ci1, cw1 = pick(a2T)
    cidx_ref[0:1, :] = ci0
    cidx_ref[1:2, :] = ci1
    cw_ref[:, 0:1] = jnp.transpose(cw0)
    cw_ref[:, 1:2] = jnp.transpose(cw1)


def _route(x2d, gwt, gbp):
    return pl.pallas_call(
        _route_kernel,
        out_shape=(
            jax.ShapeDtypeStruct((E, CPAD), jnp.int32),
            jax.ShapeDtypeStruct((2, NT), jnp.int32),
            jax.ShapeDtypeStruct((NT, 2), jnp.float32),
        ),
    )(x2d, gwt, gbp)


def _sc_gather(table, idx, rows_per_worker):
    n, d = idx.shape[0], table.shape[1]
    mesh = plsc.VectorSubcoreMesh(core_axis_name="c", subcore_axis_name="s")

    @functools.partial(
        pl.kernel, mesh=mesh,
        out_type=jax.ShapeDtypeStruct((n, d), table.dtype),
        scratch_types=[
            pltpu.VMEM((rows_per_worker,), jnp.int32),
            pltpu.VMEM((rows_per_worker, d), table.dtype),
            pltpu.SemaphoreType.DMA,
        ],
    )
    def k(table_hbm, idx_hbm, out_hbm, idx_v, rows_v, sem):
        wid = lax.axis_index("s") * _SC_NC + lax.axis_index("c")
        base = wid * rows_per_worker
        pltpu.sync_copy(idx_hbm.at[pl.ds(base, rows_per_worker)], idx_v)
        pltpu.async_copy(table_hbm.at[idx_v], rows_v, sem).wait()
        pltpu.sync_copy(rows_v, out_hbm.at[pl.ds(base, rows_per_worker)])

    return k(table, idx)


def _ffn_kernel(xe_ref, w1_ref, b1_ref, w2_ref, b2_ref, y_ref):
    f = pl.program_id(1)
    xb = xe_ref[0].astype(jnp.bfloat16)
    w1b = w1_ref[0].astype(jnp.bfloat16)
    h = lax.dot_general(
        xb, w1b, (((1,), (1,)), ((), ())), preferred_element_type=jnp.float32
    ) + b1_ref[0]
    h = 0.5 * h * (1.0 + lax.erf(h * 0.7071067811865476))
    hb = h.astype(jnp.bfloat16)
    w2b = w2_ref[0].astype(jnp.bfloat16)
    y = lax.dot_general(
        hb, w2b, (((1,), (1,)), ((), ())), preferred_element_type=jnp.float32
    )

    @pl.when(f == 0)
    def _():
        y_ref[0] = y + b2_ref[0]

    @pl.when(f != 0)
    def _():
        y_ref[0] += y


def _ffn(xe3, w1, b1, w2, b2):
    nf = DFF // FT
    b1r = b1.reshape(E * nf, 1, FT)
    b2r = b2.reshape(E, 1, DIM)
    return pl.pallas_call(
        _ffn_kernel,
        grid=(E, nf),
        in_specs=[
            pl.BlockSpec((1, CPAD, DIM), lambda e, f: (e, 0, 0)),
            pl.BlockSpec((1, FT, DIM), lambda e, f: (e, f, 0)),
            pl.BlockSpec((1, 1, FT), lambda e, f: (e * nf + f, 0, 0)),
            pl.BlockSpec((1, DIM, FT), lambda e, f: (e, 0, f)),
            pl.BlockSpec((1, 1, DIM), lambda e, f: (e, 0, 0)),
        ],
        out_specs=pl.BlockSpec((1, CPAD, DIM), lambda e, f: (e, 0, 0)),
        out_shape=jax.ShapeDtypeStruct((E, CPAD, DIM), jnp.float32),
    )(xe3, w1, b1r, w2, b2r)


def _combine_kernel(g_ref, cw_ref, o_ref):
    w0 = cw_ref[:, 0:1]
    w1 = cw_ref[:, 1:2]
    o_ref[...] = g_ref[0] * w0 + g_ref[1] * w1


def _combine(g3, cw):
    return pl.pallas_call(
        _combine_kernel,
        out_shape=jax.ShapeDtypeStruct((NT, DIM), jnp.float32),
    )(g3, cw)


def kernel(x, gate_w, gate_b, w1, b1, w2, b2):
    B, S, D = x.shape
    x2d = x.reshape(NT, DIM)
    gwt = jnp.zeros((DIM, LPAD), jnp.float32).at[:, :E].set(gate_w.T)
    gbp = jnp.zeros((1, LPAD), jnp.float32).at[0, :E].set(gate_b)

    dtok, cidx, cw = _route(x2d, gwt, gbp)
    xe = _sc_gather(x2d, dtok.reshape(NROWS), NROWS // _SC_NW)
    y = _ffn(xe.reshape(E, CPAD, DIM), w1, b1, w2, b2)
    g = _sc_gather(y.reshape(NROWS, DIM), cidx.reshape(2 * NT), 2 * NT // _SC_NW)
    out = _combine(g.reshape(2, NT, DIM), cw)
    return out.reshape(B, S, D)


# speedup vs baseline: 4.3336x; 1.3248x over previous
import functools

import jax
import jax.numpy as jnp
from jax import lax
from jax.experimental import pallas as pl
from jax.experimental.pallas import tpu as pltpu
from jax.experimental.pallas import tpu_sc as plsc

DIM = 768
DFF = 3072
E = 8
NT = 2048
CAP = 307
CPAD = 320
LPAD = 128
FT = 512
NROWS = E * CPAD

_SC_NC = 2
_SC_NS = 16
_SC_NW = _SC_NC * _SC_NS


def _lane_cumsum(a):
    n = a.shape[-1]
    sh = 1
    while sh < n:
        z = jnp.zeros(a.shape[:-1] + (sh,), a.dtype)
        a = a + jnp.concatenate([z, a[..., : n - sh]], axis=-1)
        sh *= 2
    return a


def _route_kernel(x_ref, gwt_ref, gb_ref, dtok_ref, sdst_ref, cw_ref):
    x = x_ref[...]
    logits = lax.dot_general(
        x, gwt_ref[...], (((1,), (0,)), ((), ())),
        preferred_element_type=jnp.float32,
    ) + gb_ref[...]
    lane = lax.broadcasted_iota(jnp.int32, (NT, LPAD), 1)
    logits = jnp.where(lane < E, logits, -jnp.inf)
    m = jnp.max(logits, axis=1, keepdims=True)
    el = jnp.exp(logits - m)
    p = el / jnp.sum(el, axis=1, keepdims=True)

    m1 = jnp.max(p, axis=1, keepdims=True)
    a1 = jnp.min(jnp.where(p == m1, lane, LPAD), axis=1, keepdims=True)
    p2 = jnp.where(lane == a1, -1.0, p)
    m2 = jnp.max(p2, axis=1, keepdims=True)
    a2 = jnp.min(jnp.where(p2 == m2, lane, LPAD), axis=1, keepdims=True)

    pT = jnp.transpose(p[:, :E])
    a1T = jnp.transpose(a1)
    a2T = jnp.transpose(a2)
    eio = lax.broadcasted_iota(jnp.int32, (E, NT), 0)
    sel = (eio == a1T) | (eio == a2T)

    keys = jnp.where(sel, lax.bitcast_convert_type(pT, jnp.int32), -1)

    lo = jnp.zeros((E, 1), jnp.int32)
    hi = jnp.full((E, 1), 1 << 30, jnp.int32)
    for _ in range(31):
        mid = lo + ((hi - lo + 1) >> 1)
        cnt = jnp.sum((keys >= mid).astype(jnp.int32), axis=1, keepdims=True)
        pred = cnt >= CAP
        lo = jnp.where(pred, mid, lo)
        hi = jnp.where(pred, hi, mid - 1)
    kept = keys >= lo

    cums = _lane_cumsum(kept.astype(jnp.int32))
    slot = cums - 1
    kvec = (eio != a1T).astype(jnp.int32)

    cio = lax.broadcasted_iota(jnp.int32, (CPAD, NT), 0)
    tio = lax.broadcasted_iota(jnp.int32, (1, NT), 1)
    for e in range(E):
        onehot = ((slot[e : e + 1, :] == cio) & kept[e : e + 1, :]).astype(
            jnp.int32
        )
        t_star = jnp.sum(onehot * tio, axis=1)
        k_sel = jnp.sum(onehot * kvec[e : e + 1, :], axis=1)
        filled = jnp.sum(onehot, axis=1) > 0
        dtok_ref[e, :] = jnp.where(filled, t_star, 0)
        sdst_ref[e, :] = jnp.where(filled, k_sel * NT + t_star, 2 * NT)

    def pick(aT):
        onehot = eio == aT
        kp = jnp.sum(jnp.where(onehot & kept, 1, 0), axis=0, keepdims=True) > 0
        wv = jnp.sum(jnp.where(onehot, pT, 0.0), axis=0, keepdims=True)
        return jnp.where(kp, wv, 0.0)

    cw_ref[:, 0:1] = jnp.transpose(pick(a1T))
    cw_ref[:, 1:2] = jnp.transpose(pick(a2T))


def _route(x2d, gwt, gbp):
    return pl.pallas_call(
        _route_kernel,
        out_shape=(
            jax.ShapeDtypeStruct((E, CPAD), jnp.int32),
            jax.ShapeDtypeStruct((E, CPAD), jnp.int32),
            jax.ShapeDtypeStruct((NT, 2), jnp.float32),
        ),
    )(x2d, gwt, gbp)


def _sc_gather(table, idx, rows_per_worker):
    n, d = idx.shape[0], table.shape[1]
    mesh = plsc.VectorSubcoreMesh(core_axis_name="c", subcore_axis_name="s")

    @functools.partial(
        pl.kernel, mesh=mesh,
        out_type=jax.ShapeDtypeStruct((n, d), table.dtype),
        scratch_types=[
            pltpu.VMEM((rows_per_worker,), jnp.int32),
            pltpu.VMEM((rows_per_worker, d), table.dtype),
            pltpu.SemaphoreType.DMA,
        ],
    )
    def k(table_hbm, idx_hbm, out_hbm, idx_v, rows_v, sem):
        wid = lax.axis_index("s") * _SC_NC + lax.axis_index("c")
        base = wid * rows_per_worker
        pltpu.sync_copy(idx_hbm.at[pl.ds(base, rows_per_worker)], idx_v)
        pltpu.async_copy(table_hbm.at[idx_v], rows_v, sem).wait()
        pltpu.sync_copy(rows_v, out_hbm.at[pl.ds(base, rows_per_worker)])

    return k(table, idx)


def _sc_scatter(rows, dst, n_out, rows_per_worker):
    d = rows.shape[1]
    mesh = plsc.VectorSubcoreMesh(core_axis_name="c", subcore_axis_name="s")

    @functools.partial(
        pl.kernel, mesh=mesh,
        out_type=jax.ShapeDtypeStruct((n_out, d), rows.dtype),
        scratch_types=[
            pltpu.VMEM((rows_per_worker,), jnp.int32),
            pltpu.VMEM((rows_per_worker, d), rows.dtype),
            pltpu.SemaphoreType.DMA,
        ],
    )
    def k(rows_hbm, dst_hbm, out_hbm, dst_v, rows_v, sem):
        wid = lax.axis_index("s") * _SC_NC + lax.axis_index("c")
        base = wid * rows_per_worker
        pltpu.sync_copy(dst_hbm.at[pl.ds(base, rows_per_worker)], dst_v)
        pltpu.sync_copy(rows_hbm.at[pl.ds(base, rows_per_worker)], rows_v)
        pltpu.async_copy(rows_v, out_hbm.at[dst_v], sem).wait()

    return k(rows, dst)


def _ffn_kernel(xe_ref, w1_ref, b1_ref, w2_ref, b2_ref, y_ref):
    f = pl.program_id(1)
    xb = xe_ref[0].astype(jnp.bfloat16)
    w1b = w1_ref[0].astype(jnp.bfloat16)
    h = lax.dot_general(
        xb, w1b, (((1,), (1,)), ((), ())), preferred_element_type=jnp.float32
    ) + b1_ref[0]
    h = 0.5 * h * (1.0 + lax.erf(h * 0.7071067811865476))
    hb = h.astype(jnp.bfloat16)
    w2b = w2_ref[0].astype(jnp.bfloat16)
    y = lax.dot_general(
        hb, w2b, (((1,), (1,)), ((), ())), preferred_element_type=jnp.float32
    )

    @pl.when(f == 0)
    def _():
        y_ref[0] = y + b2_ref[0]

    @pl.when(f != 0)
    def _():
        y_ref[0] += y


def _ffn(xe3, w1, b1, w2, b2):
    nf = DFF // FT
    b1r = b1.reshape(E * nf, 1, FT)
    b2r = b2.reshape(E, 1, DIM)
    return pl.pallas_call(
        _ffn_kernel,
        grid=(E, nf),
        in_specs=[
            pl.BlockSpec((1, CPAD, DIM), lambda e, f: (e, 0, 0)),
            pl.BlockSpec((1, FT, DIM), lambda e, f: (e, f, 0)),
            pl.BlockSpec((1, 1, FT), lambda e, f: (e * nf + f, 0, 0)),
            pl.BlockSpec((1, DIM, FT), lambda e, f: (e, 0, f)),
            pl.BlockSpec((1, 1, DIM), lambda e, f: (e, 0, 0)),
        ],
        out_specs=pl.BlockSpec((1, CPAD, DIM), lambda e, f: (e, 0, 0)),
        out_shape=jax.ShapeDtypeStruct((E, CPAD, DIM), jnp.float32),
    )(xe3, w1, b1r, w2, b2r)


def _combine_kernel(g_ref, cw_ref, o_ref):
    w0 = cw_ref[:, 0:1]
    w1 = cw_ref[:, 1:2]
    c0 = jnp.where(w0 > 0, g_ref[0] * w0, 0.0)
    c1 = jnp.where(w1 > 0, g_ref[1] * w1, 0.0)
    o_ref[...] = c0 + c1


def _combine(g3, cw):
    return pl.pallas_call(
        _combine_kernel,
        out_shape=jax.ShapeDtypeStruct((NT, DIM), jnp.float32),
    )(g3, cw)


def kernel(x, gate_w, gate_b, w1, b1, w2, b2):
    B, S, D = x.shape
    x2d = x.reshape(NT, DIM)
    gwt = jnp.zeros((DIM, LPAD), jnp.float32).at[:, :E].set(gate_w.T)
    gbp = jnp.zeros((1, LPAD), jnp.float32).at[0, :E].set(gate_b)

    dtok, sdst, cw = _route(x2d, gwt, gbp)
    xe = _sc_gather(x2d, dtok.reshape(NROWS), NROWS // _SC_NW)
    y = _ffn(xe.reshape(E, CPAD, DIM), w1, b1, w2, b2)
    g = _sc_scatter(y.reshape(NROWS, DIM), sdst.reshape(NROWS),
                    2 * NT + 8, NROWS // _SC_NW)
    out = _combine(g[: 2 * NT].reshape(2, NT, DIM), cw)
    return out.reshape(B, S, D)


# speedup vs baseline: 4.8470x; 1.1185x over previous
import functools

import jax
import jax.numpy as jnp
from jax import lax
from jax.experimental import pallas as pl
from jax.experimental.pallas import tpu as pltpu
from jax.experimental.pallas import tpu_sc as plsc

DIM = 768
DFF = 3072
E = 8
NT = 2048
CAP = 307
CPAD = 320
LPAD = 128
FT = 1024
NROWS = E * CPAD

_SC_NC = 2
_SC_NS = 16
_SC_NW = _SC_NC * _SC_NS


def _lane_cumsum(a):
    n = a.shape[-1]
    sh = 1
    while sh < n:
        z = jnp.zeros(a.shape[:-1] + (sh,), a.dtype)
        a = a + jnp.concatenate([z, a[..., : n - sh]], axis=-1)
        sh *= 2
    return a


def _route_kernel(x_ref, gwt_ref, gb_ref, dtok_ref, sdst_ref, cw_ref):
    x = x_ref[...]
    logits = lax.dot_general(
        x, gwt_ref[...], (((1,), (0,)), ((), ())),
        preferred_element_type=jnp.float32,
    ) + gb_ref[...]
    lane = lax.broadcasted_iota(jnp.int32, (NT, LPAD), 1)
    logits = jnp.where(lane < E, logits, -jnp.inf)
    m = jnp.max(logits, axis=1, keepdims=True)
    el = jnp.exp(logits - m)
    p = el / jnp.sum(el, axis=1, keepdims=True)

    m1 = jnp.max(p, axis=1, keepdims=True)
    a1 = jnp.min(jnp.where(p == m1, lane, LPAD), axis=1, keepdims=True)
    p2 = jnp.where(lane == a1, -1.0, p)
    m2 = jnp.max(p2, axis=1, keepdims=True)
    a2 = jnp.min(jnp.where(p2 == m2, lane, LPAD), axis=1, keepdims=True)

    pT = jnp.transpose(p[:, :E])
    a1T = jnp.transpose(a1)
    a2T = jnp.transpose(a2)
    eio = lax.broadcasted_iota(jnp.int32, (E, NT), 0)
    sel = (eio == a1T) | (eio == a2T)

    keys = jnp.where(sel, lax.bitcast_convert_type(pT, jnp.int32), -1)

    lo = jnp.zeros((E, 1), jnp.int32)
    hi = jnp.full((E, 1), 1 << 30, jnp.int32)
    for _ in range(31):
        mid = lo + ((hi - lo + 1) >> 1)
        cnt = jnp.sum((keys >= mid).astype(jnp.int32), axis=1, keepdims=True)
        pred = cnt >= CAP
        lo = jnp.where(pred, mid, lo)
        hi = jnp.where(pred, hi, mid - 1)
    kept = keys >= lo

    cums = _lane_cumsum(kept.astype(jnp.int32))
    slot = cums - 1
    kvec = (eio != a1T).astype(jnp.int32)

    cio = lax.broadcasted_iota(jnp.int32, (CPAD, NT), 0)
    tio = lax.broadcasted_iota(jnp.int32, (1, NT), 1)
    for e in range(E):
        onehot = ((slot[e : e + 1, :] == cio) & kept[e : e + 1, :]).astype(
            jnp.int32
        )
        t_star = jnp.sum(onehot * tio, axis=1)
        k_sel = jnp.sum(onehot * kvec[e : e + 1, :], axis=1)
        filled = jnp.sum(onehot, axis=1) > 0
        dtok_ref[e, :] = jnp.where(filled, t_star, 0)
        sdst_ref[e, :] = jnp.where(filled, k_sel * NT + t_star, 2 * NT)

    def pick(aT):
        onehot = eio == aT
        kp = jnp.sum(jnp.where(onehot & kept, 1, 0), axis=0, keepdims=True) > 0
        wv = jnp.sum(jnp.where(onehot, pT, 0.0), axis=0, keepdims=True)
        return jnp.where(kp, wv, 0.0)

    cw_ref[:, 0:1] = jnp.transpose(pick(a1T))
    cw_ref[:, 1:2] = jnp.transpose(pick(a2T))


def _route(x2d, gwt, gbp):
    return pl.pallas_call(
        _route_kernel,
        out_shape=(
            jax.ShapeDtypeStruct((E, CPAD), jnp.int32),
            jax.ShapeDtypeStruct((E, CPAD), jnp.int32),
            jax.ShapeDtypeStruct((NT, 2), jnp.float32),
        ),
    )(x2d, gwt, gbp)


def _sc_gather(table, idx, rows_per_worker):
    n, d = idx.shape[0], table.shape[1]
    mesh = plsc.VectorSubcoreMesh(core_axis_name="c", subcore_axis_name="s")

    @functools.partial(
        pl.kernel, mesh=mesh,
        out_type=jax.ShapeDtypeStruct((n, d), table.dtype),
        scratch_types=[
            pltpu.VMEM((rows_per_worker,), jnp.int32),
            pltpu.VMEM((rows_per_worker, d), table.dtype),
            pltpu.SemaphoreType.DMA,
        ],
    )
    def k(table_hbm, idx_hbm, out_hbm, idx_v, rows_v, sem):
        wid = lax.axis_index("s") * _SC_NC + lax.axis_index("c")
        base = wid * rows_per_worker
        pltpu.sync_copy(idx_hbm.at[pl.ds(base, rows_per_worker)], idx_v)
        pltpu.async_copy(table_hbm.at[idx_v], rows_v, sem).wait()
        pltpu.sync_copy(rows_v, out_hbm.at[pl.ds(base, rows_per_worker)])

    return k(table, idx)


def _sc_scatter(rows, dst, n_out, rows_per_worker):
    d = rows.shape[1]
    mesh = plsc.VectorSubcoreMesh(core_axis_name="c", subcore_axis_name="s")

    @functools.partial(
        pl.kernel, mesh=mesh,
        out_type=jax.ShapeDtypeStruct((n_out, d), rows.dtype),
        scratch_types=[
            pltpu.VMEM((rows_per_worker,), jnp.int32),
            pltpu.VMEM((rows_per_worker, d), rows.dtype),
            pltpu.SemaphoreType.DMA,
        ],
    )
    def k(rows_hbm, dst_hbm, out_hbm, dst_v, rows_v, sem):
        wid = lax.axis_index("s") * _SC_NC + lax.axis_index("c")
        base = wid * rows_per_worker
        pltpu.sync_copy(dst_hbm.at[pl.ds(base, rows_per_worker)], dst_v)
        pltpu.sync_copy(rows_hbm.at[pl.ds(base, rows_per_worker)], rows_v)
        pltpu.async_copy(rows_v, out_hbm.at[dst_v], sem).wait()

    return k(rows, dst)


def _ffn_kernel(xe_ref, w1_ref, b1_ref, w2_ref, b2_ref, y_ref):
    f = pl.program_id(1)
    xb = xe_ref[0].astype(jnp.bfloat16)
    w1b = w1_ref[0].astype(jnp.bfloat16)
    h = lax.dot_general(
        xb, w1b, (((1,), (1,)), ((), ())), preferred_element_type=jnp.float32
    ) + b1_ref[0]
    h = 0.5 * h * (1.0 + lax.erf(h * 0.7071067811865476))
    hb = h.astype(jnp.bfloat16)
    w2b = w2_ref[0].astype(jnp.bfloat16)
    y = lax.dot_general(
        hb, w2b, (((1,), (1,)), ((), ())), preferred_element_type=jnp.float32
    )

    @pl.when(f == 0)
    def _():
        y_ref[0] = y + b2_ref[0]

    @pl.when(f != 0)
    def _():
        y_ref[0] += y


def _ffn(xe3, w1, b1, w2, b2):
    nf = DFF // FT
    b1r = b1.reshape(E * nf, 1, FT)
    b2r = b2.reshape(E, 1, DIM)
    return pl.pallas_call(
        _ffn_kernel,
        grid=(E, nf),
        in_specs=[
            pl.BlockSpec((1, CPAD, DIM), lambda e, f: (e, 0, 0)),
            pl.BlockSpec((1, FT, DIM), lambda e, f: (e, f, 0)),
            pl.BlockSpec((1, 1, FT), lambda e, f: (e * nf + f, 0, 0)),
            pl.BlockSpec((1, DIM, FT), lambda e, f: (e, 0, f)),
            pl.BlockSpec((1, 1, DIM), lambda e, f: (e, 0, 0)),
        ],
        out_specs=pl.BlockSpec((1, CPAD, DIM), lambda e, f: (e, 0, 0)),
        out_shape=jax.ShapeDtypeStruct((E, CPAD, DIM), jnp.float32),
    )(xe3, w1, b1r, w2, b2r)


def _combine_kernel(g_ref, cw_ref, o_ref):
    w0 = cw_ref[:, 0:1]
    w1 = cw_ref[:, 1:2]
    c0 = jnp.where(w0 > 0, g_ref[0] * w0, 0.0)
    c1 = jnp.where(w1 > 0, g_ref[1] * w1, 0.0)
    o_ref[...] = c0 + c1


def _combine(g3, cw):
    return pl.pallas_call(
        _combine_kernel,
        out_shape=jax.ShapeDtypeStruct((NT, DIM), jnp.float32),
    )(g3, cw)


def kernel(x, gate_w, gate_b, w1, b1, w2, b2):
    B, S, D = x.shape
    x2d = x.reshape(NT, DIM)
    gwt = jnp.zeros((DIM, LPAD), jnp.float32).at[:, :E].set(gate_w.T)
    gbp = jnp.zeros((1, LPAD), jnp.float32).at[0, :E].set(gate_b)

    dtok, sdst, cw = _route(x2d, gwt, gbp)
    xe = _sc_gather(x2d, dtok.reshape(NROWS), NROWS // _SC_NW)
    y = _ffn(xe.reshape(E, CPAD, DIM), w1, b1, w2, b2)
    g = _sc_scatter(y.reshape(NROWS, DIM), sdst.reshape(NROWS),
                    2 * NT + 8, NROWS // _SC_NW)
    out = _combine(g[: 2 * NT].reshape(2, NT, DIM), cw)
    return out.reshape(B, S, D)


# speedup vs baseline: 4.9505x; 1.0214x over previous
import functools

import jax
import jax.numpy as jnp
from jax import lax
from jax.experimental import pallas as pl
from jax.experimental.pallas import tpu as pltpu
from jax.experimental.pallas import tpu_sc as plsc

DIM = 768
DFF = 3072
E = 8
NT = 2048
CAP = 307
CPAD = 320
LPAD = 128
FT = 1536
NROWS = E * CPAD

_SC_NC = 2
_SC_NS = 16
_SC_NW = _SC_NC * _SC_NS


def _lane_cumsum(a):
    n = a.shape[-1]
    sh = 1
    while sh < n:
        z = jnp.zeros(a.shape[:-1] + (sh,), a.dtype)
        a = a + jnp.concatenate([z, a[..., : n - sh]], axis=-1)
        sh *= 2
    return a


def _route_kernel(x_ref, gwt_ref, gb_ref, dtok_ref, sdst_ref, cw_ref):
    x = x_ref[...]
    logits = lax.dot_general(
        x, gwt_ref[...], (((1,), (0,)), ((), ())),
        preferred_element_type=jnp.float32,
    ) + gb_ref[...]
    lane = lax.broadcasted_iota(jnp.int32, (NT, LPAD), 1)
    logits = jnp.where(lane < E, logits, -jnp.inf)
    m = jnp.max(logits, axis=1, keepdims=True)
    el = jnp.exp(logits - m)
    p = el / jnp.sum(el, axis=1, keepdims=True)

    m1 = jnp.max(p, axis=1, keepdims=True)
    a1 = jnp.min(jnp.where(p == m1, lane, LPAD), axis=1, keepdims=True)
    p2 = jnp.where(lane == a1, -1.0, p)
    m2 = jnp.max(p2, axis=1, keepdims=True)
    a2 = jnp.min(jnp.where(p2 == m2, lane, LPAD), axis=1, keepdims=True)

    pT = jnp.transpose(p[:, :E])
    a1T = jnp.transpose(a1)
    a2T = jnp.transpose(a2)
    eio = lax.broadcasted_iota(jnp.int32, (E, NT), 0)
    sel = (eio == a1T) | (eio == a2T)

    keys = jnp.where(sel, lax.bitcast_convert_type(pT, jnp.int32), -1)

    lo = jnp.zeros((E, 1), jnp.int32)
    hi = jnp.full((E, 1), 1 << 30, jnp.int32)
    for _ in range(31):
        mid = lo + ((hi - lo + 1) >> 1)
        cnt = jnp.sum((keys >= mid).astype(jnp.int32), axis=1, keepdims=True)
        pred = cnt >= CAP
        lo = jnp.where(pred, mid, lo)
        hi = jnp.where(pred, hi, mid - 1)
    kept = keys >= lo

    cums = _lane_cumsum(kept.astype(jnp.int32))
    slot = cums - 1
    kvec = (eio != a1T).astype(jnp.int32)

    cio = lax.broadcasted_iota(jnp.int32, (CPAD, NT), 0)
    tio = lax.broadcasted_iota(jnp.int32, (1, NT), 1)
    for e in range(E):
        onehot = ((slot[e : e + 1, :] == cio) & kept[e : e + 1, :]).astype(
            jnp.int32
        )
        t_star = jnp.sum(onehot * tio, axis=1)
        k_sel = jnp.sum(onehot * kvec[e : e + 1, :], axis=1)
        filled = jnp.sum(onehot, axis=1) > 0
        dtok_ref[e, :] = jnp.where(filled, t_star, 0)
        sdst_ref[e, :] = jnp.where(filled, k_sel * NT + t_star, 2 * NT)

    def pick(aT):
        onehot = eio == aT
        kp = jnp.sum(jnp.where(onehot & kept, 1, 0), axis=0, keepdims=True) > 0
        wv = jnp.sum(jnp.where(onehot, pT, 0.0), axis=0, keepdims=True)
        return jnp.where(kp, wv, 0.0)

    cw_ref[:, 0:1] = jnp.transpose(pick(a1T))
    cw_ref[:, 1:2] = jnp.transpose(pick(a2T))


def _route(x2d, gwt, gbp):
    return pl.pallas_call(
        _route_kernel,
        out_shape=(
            jax.ShapeDtypeStruct((E, CPAD), jnp.int32),
            jax.ShapeDtypeStruct((E, CPAD), jnp.int32),
            jax.ShapeDtypeStruct((NT, 2), jnp.float32),
        ),
    )(x2d, gwt, gbp)


def _sc_gather(table, idx, rows_per_worker):
    n, d = idx.shape[0], table.shape[1]
    mesh = plsc.VectorSubcoreMesh(core_axis_name="c", subcore_axis_name="s")

    @functools.partial(
        pl.kernel, mesh=mesh,
        out_type=jax.ShapeDtypeStruct((n, d), table.dtype),
        scratch_types=[
            pltpu.VMEM((rows_per_worker,), jnp.int32),
            pltpu.VMEM((rows_per_worker, d), table.dtype),
            pltpu.SemaphoreType.DMA,
        ],
    )
    def k(table_hbm, idx_hbm, out_hbm, idx_v, rows_v, sem):
        wid = lax.axis_index("s") * _SC_NC + lax.axis_index("c")
        base = wid * rows_per_worker
        pltpu.sync_copy(idx_hbm.at[pl.ds(base, rows_per_worker)], idx_v)
        pltpu.async_copy(table_hbm.at[idx_v], rows_v, sem).wait()
        pltpu.sync_copy(rows_v, out_hbm.at[pl.ds(base, rows_per_worker)])

    return k(table, idx)


def _sc_scatter(rows, dst, n_out, rows_per_worker):
    d = rows.shape[1]
    mesh = plsc.VectorSubcoreMesh(core_axis_name="c", subcore_axis_name="s")

    @functools.partial(
        pl.kernel, mesh=mesh,
        out_type=jax.ShapeDtypeStruct((n_out, d), rows.dtype),
        scratch_types=[
            pltpu.VMEM((rows_per_worker,), jnp.int32),
            pltpu.VMEM((rows_per_worker, d), rows.dtype),
            pltpu.SemaphoreType.DMA,
        ],
    )
    def k(rows_hbm, dst_hbm, out_hbm, dst_v, rows_v, sem):
        wid = lax.axis_index("s") * _SC_NC + lax.axis_index("c")
        base = wid * rows_per_worker
        pltpu.sync_copy(dst_hbm.at[pl.ds(base, rows_per_worker)], dst_v)
        pltpu.sync_copy(rows_hbm.at[pl.ds(base, rows_per_worker)], rows_v)
        pltpu.async_copy(rows_v, out_hbm.at[dst_v], sem).wait()

    return k(rows, dst)


def _ffn_kernel(xe_ref, w1_ref, b1_ref, w2_ref, b2_ref, y_ref):
    f = pl.program_id(1)
    xb = xe_ref[0].astype(jnp.bfloat16)
    w1b = w1_ref[0].astype(jnp.bfloat16)
    h = lax.dot_general(
        xb, w1b, (((1,), (1,)), ((), ())), preferred_element_type=jnp.float32
    ) + b1_ref[0]
    h = 0.5 * h * (1.0 + lax.erf(h * 0.7071067811865476))
    hb = h.astype(jnp.bfloat16)
    w2b = w2_ref[0].astype(jnp.bfloat16)
    y = lax.dot_general(
        hb, w2b, (((1,), (1,)), ((), ())), preferred_element_type=jnp.float32
    )

    @pl.when(f == 0)
    def _():
        y_ref[0] = y + b2_ref[0]

    @pl.when(f != 0)
    def _():
        y_ref[0] += y


def _ffn(xe3, w1, b1, w2, b2):
    nf = DFF // FT
    b1r = b1.reshape(E * nf, 1, FT)
    b2r = b2.reshape(E, 1, DIM)
    return pl.pallas_call(
        _ffn_kernel,
        grid=(E, nf),
        in_specs=[
            pl.BlockSpec((1, CPAD, DIM), lambda e, f: (e, 0, 0)),
            pl.BlockSpec((1, FT, DIM), lambda e, f: (e, f, 0)),
            pl.BlockSpec((1, 1, FT), lambda e, f: (e * nf + f, 0, 0)),
            pl.BlockSpec((1, DIM, FT), lambda e, f: (e, 0, f)),
            pl.BlockSpec((1, 1, DIM), lambda e, f: (e, 0, 0)),
        ],
        out_specs=pl.BlockSpec((1, CPAD, DIM), lambda e, f: (e, 0, 0)),
        out_shape=jax.ShapeDtypeStruct((E, CPAD, DIM), jnp.float32),
    )(xe3, w1, b1r, w2, b2r)


def _combine_kernel(g_ref, cw_ref, o_ref):
    w0 = cw_ref[:, 0:1]
    w1 = cw_ref[:, 1:2]
    c0 = jnp.where(w0 > 0, g_ref[0] * w0, 0.0)
    c1 = jnp.where(w1 > 0, g_ref[1] * w1, 0.0)
    o_ref[...] = c0 + c1


def _combine(g3, cw):
    return pl.pallas_call(
        _combine_kernel,
        out_shape=jax.ShapeDtypeStruct((NT, DIM), jnp.float32),
    )(g3, cw)


def kernel(x, gate_w, gate_b, w1, b1, w2, b2):
    B, S, D = x.shape
    x2d = x.reshape(NT, DIM)
    gwt = jnp.zeros((DIM, LPAD), jnp.float32).at[:, :E].set(gate_w.T)
    gbp = jnp.zeros((1, LPAD), jnp.float32).at[0, :E].set(gate_b)

    dtok, sdst, cw = _route(x2d, gwt, gbp)
    xe = _sc_gather(x2d, dtok.reshape(NROWS), NROWS // _SC_NW)
    y = _ffn(xe.reshape(E, CPAD, DIM), w1, b1, w2, b2)
    g = _sc_scatter(y.reshape(NROWS, DIM), sdst.reshape(NROWS),
                    2 * NT + 8, NROWS // _SC_NW)
    out = _combine(g[: 2 * NT].reshape(2, NT, DIM), cw)
    return out.reshape(B, S, D)


# speedup vs baseline: 5.0162x; 1.0133x over previous
import functools

import jax
import jax.numpy as jnp
from jax import lax
from jax.experimental import pallas as pl
from jax.experimental.pallas import tpu as pltpu
from jax.experimental.pallas import tpu_sc as plsc

DIM = 768
DFF = 3072
E = 8
NT = 2048
CAP = 307
CPAD = 320
LPAD = 128
FT = 3072
NROWS = E * CPAD

_SC_NC = 2
_SC_NS = 16
_SC_NW = _SC_NC * _SC_NS


def _lane_cumsum(a):
    n = a.shape[-1]
    sh = 1
    while sh < n:
        z = jnp.zeros(a.shape[:-1] + (sh,), a.dtype)
        a = a + jnp.concatenate([z, a[..., : n - sh]], axis=-1)
        sh *= 2
    return a


def _route_kernel(x_ref, gwt_ref, gb_ref, dtok_ref, sdst_ref, cw_ref):
    x = x_ref[...]
    logits = lax.dot_general(
        x, gwt_ref[...], (((1,), (0,)), ((), ())),
        preferred_element_type=jnp.float32,
    ) + gb_ref[...]
    lane = lax.broadcasted_iota(jnp.int32, (NT, LPAD), 1)
    logits = jnp.where(lane < E, logits, -jnp.inf)
    m = jnp.max(logits, axis=1, keepdims=True)
    el = jnp.exp(logits - m)
    p = el / jnp.sum(el, axis=1, keepdims=True)

    m1 = jnp.max(p, axis=1, keepdims=True)
    a1 = jnp.min(jnp.where(p == m1, lane, LPAD), axis=1, keepdims=True)
    p2 = jnp.where(lane == a1, -1.0, p)
    m2 = jnp.max(p2, axis=1, keepdims=True)
    a2 = jnp.min(jnp.where(p2 == m2, lane, LPAD), axis=1, keepdims=True)

    pT = jnp.transpose(p[:, :E])
    a1T = jnp.transpose(a1)
    a2T = jnp.transpose(a2)
    eio = lax.broadcasted_iota(jnp.int32, (E, NT), 0)
    sel = (eio == a1T) | (eio == a2T)

    keys = jnp.where(sel, lax.bitcast_convert_type(pT, jnp.int32), -1)

    lo = jnp.zeros((E, 1), jnp.int32)
    hi = jnp.full((E, 1), 1 << 30, jnp.int32)
    for _ in range(31):
        mid = lo + ((hi - lo + 1) >> 1)
        cnt = jnp.sum((keys >= mid).astype(jnp.int32), axis=1, keepdims=True)
        pred = cnt >= CAP
        lo = jnp.where(pred, mid, lo)
        hi = jnp.where(pred, hi, mid - 1)
    kept = keys >= lo

    cums = _lane_cumsum(kept.astype(jnp.int32))
    slot = cums - 1
    kvec = (eio != a1T).astype(jnp.int32)

    cio = lax.broadcasted_iota(jnp.int32, (CPAD, NT), 0)
    tio = lax.broadcasted_iota(jnp.int32, (1, NT), 1)
    for e in range(E):
        onehot = ((slot[e : e + 1, :] == cio) & kept[e : e + 1, :]).astype(
            jnp.int32
        )
        t_star = jnp.sum(onehot * tio, axis=1)
        k_sel = jnp.sum(onehot * kvec[e : e + 1, :], axis=1)
        filled = jnp.sum(onehot, axis=1) > 0
        dtok_ref[e, :] = jnp.where(filled, t_star, 0)
        sdst_ref[e, :] = jnp.where(filled, k_sel * NT + t_star, 2 * NT)

    def pick(aT):
        onehot = eio == aT
        kp = jnp.sum(jnp.where(onehot & kept, 1, 0), axis=0, keepdims=True) > 0
        wv = jnp.sum(jnp.where(onehot, pT, 0.0), axis=0, keepdims=True)
        return jnp.where(kp, wv, 0.0)

    cw_ref[:, 0:1] = jnp.transpose(pick(a1T))
    cw_ref[:, 1:2] = jnp.transpose(pick(a2T))


def _route(x2d, gwt, gbp):
    return pl.pallas_call(
        _route_kernel,
        out_shape=(
            jax.ShapeDtypeStruct((E, CPAD), jnp.int32),
            jax.ShapeDtypeStruct((E, CPAD), jnp.int32),
            jax.ShapeDtypeStruct((NT, 2), jnp.float32),
        ),
    )(x2d, gwt, gbp)


def _sc_gather(table, idx, rows_per_worker):
    n, d = idx.shape[0], table.shape[1]
    mesh = plsc.VectorSubcoreMesh(core_axis_name="c", subcore_axis_name="s")

    @functools.partial(
        pl.kernel, mesh=mesh,
        out_type=jax.ShapeDtypeStruct((n, d), table.dtype),
        scratch_types=[
            pltpu.VMEM((rows_per_worker,), jnp.int32),
            pltpu.VMEM((rows_per_worker, d), table.dtype),
            pltpu.SemaphoreType.DMA,
        ],
    )
    def k(table_hbm, idx_hbm, out_hbm, idx_v, rows_v, sem):
        wid = lax.axis_index("s") * _SC_NC + lax.axis_index("c")
        base = wid * rows_per_worker
        pltpu.sync_copy(idx_hbm.at[pl.ds(base, rows_per_worker)], idx_v)
        pltpu.async_copy(table_hbm.at[idx_v], rows_v, sem).wait()
        pltpu.sync_copy(rows_v, out_hbm.at[pl.ds(base, rows_per_worker)])

    return k(table, idx)


def _sc_scatter(rows, dst, n_out, rows_per_worker):
    d = rows.shape[1]
    mesh = plsc.VectorSubcoreMesh(core_axis_name="c", subcore_axis_name="s")

    @functools.partial(
        pl.kernel, mesh=mesh,
        out_type=jax.ShapeDtypeStruct((n_out, d), rows.dtype),
        scratch_types=[
            pltpu.VMEM((rows_per_worker,), jnp.int32),
            pltpu.VMEM((rows_per_worker, d), rows.dtype),
            pltpu.SemaphoreType.DMA,
        ],
    )
    def k(rows_hbm, dst_hbm, out_hbm, dst_v, rows_v, sem):
        wid = lax.axis_index("s") * _SC_NC + lax.axis_index("c")
        base = wid * rows_per_worker
        pltpu.sync_copy(dst_hbm.at[pl.ds(base, rows_per_worker)], dst_v)
        pltpu.sync_copy(rows_hbm.at[pl.ds(base, rows_per_worker)], rows_v)
        pltpu.async_copy(rows_v, out_hbm.at[dst_v], sem).wait()

    return k(rows, dst)


def _ffn_kernel(xe_ref, w1_ref, b1_ref, w2_ref, b2_ref, y_ref):
    f = pl.program_id(1)
    xb = xe_ref[0].astype(jnp.bfloat16)
    w1b = w1_ref[0].astype(jnp.bfloat16)
    h = lax.dot_general(
        xb, w1b, (((1,), (1,)), ((), ())), preferred_element_type=jnp.float32
    ) + b1_ref[0]
    h = 0.5 * h * (1.0 + lax.erf(h * 0.7071067811865476))
    hb = h.astype(jnp.bfloat16)
    w2b = w2_ref[0].astype(jnp.bfloat16)
    y = lax.dot_general(
        hb, w2b, (((1,), (1,)), ((), ())), preferred_element_type=jnp.float32
    )

    @pl.when(f == 0)
    def _():
        y_ref[0] = y + b2_ref[0]

    @pl.when(f != 0)
    def _():
        y_ref[0] += y


def _ffn(xe3, w1, b1, w2, b2):
    nf = DFF // FT
    b1r = b1.reshape(E * nf, 1, FT)
    b2r = b2.reshape(E, 1, DIM)
    return pl.pallas_call(
        _ffn_kernel,
        grid=(E, nf),
        in_specs=[
            pl.BlockSpec((1, CPAD, DIM), lambda e, f: (e, 0, 0)),
            pl.BlockSpec((1, FT, DIM), lambda e, f: (e, f, 0)),
            pl.BlockSpec((1, 1, FT), lambda e, f: (e * nf + f, 0, 0)),
            pl.BlockSpec((1, DIM, FT), lambda e, f: (e, 0, f)),
            pl.BlockSpec((1, 1, DIM), lambda e, f: (e, 0, 0)),
        ],
        out_specs=pl.BlockSpec((1, CPAD, DIM), lambda e, f: (e, 0, 0)),
        out_shape=jax.ShapeDtypeStruct((E, CPAD, DIM), jnp.float32),
    )(xe3, w1, b1r, w2, b2r)


def _combine_kernel(g_ref, cw_ref, o_ref):
    w0 = cw_ref[:, 0:1]
    w1 = cw_ref[:, 1:2]
    c0 = jnp.where(w0 > 0, g_ref[0] * w0, 0.0)
    c1 = jnp.where(w1 > 0, g_ref[1] * w1, 0.0)
    o_ref[...] = c0 + c1


def _combine(g3, cw):
    return pl.pallas_call(
        _combine_kernel,
        out_shape=jax.ShapeDtypeStruct((NT, DIM), jnp.float32),
    )(g3, cw)


def kernel(x, gate_w, gate_b, w1, b1, w2, b2):
    B, S, D = x.shape
    x2d = x.reshape(NT, DIM)
    gwt = jnp.zeros((DIM, LPAD), jnp.float32).at[:, :E].set(gate_w.T)
    gbp = jnp.zeros((1, LPAD), jnp.float32).at[0, :E].set(gate_b)

    dtok, sdst, cw = _route(x2d, gwt, gbp)
    xe = _sc_gather(x2d, dtok.reshape(NROWS), NROWS // _SC_NW)
    y = _ffn(xe.reshape(E, CPAD, DIM), w1, b1, w2, b2)
    g = _sc_scatter(y.reshape(NROWS, DIM), sdst.reshape(NROWS),
                    2 * NT + 8, NROWS // _SC_NW)
    out = _combine(g[: 2 * NT].reshape(2, NT, DIM), cw)
    return out.reshape(B, S, D)
